# baseline (speedup 1.0000x reference)
import jax
import jax.numpy as jnp
from jax import lax
from jax.experimental import pallas as pl
from jax.experimental.pallas import tpu as pltpu
from jax.experimental.pallas import tpu_sc as plsc

B, L = 16384, 200
DAY_SIZE, TIME_SIZE = 32, 64
OUT = DAY_SIZE + TIME_SIZE

BHG = 32
LCH = 5
LB = L // LCH
BB = B // BHG


def _cidx_kernel(dt_ref, out_ref):
    li = jax.lax.broadcasted_iota(jnp.int32, (L, L), 0)
    lj = jax.lax.broadcasted_iota(jnp.int32, (L, L), 1)
    ident = (li == lj).astype(jnp.float32)
    dnT = (((0,), (0,)), ((), ()))
    parts = []
    for bh in range(4):
        d = dt_ref[:, 2 * bh, :]
        t = dt_ref[:, 2 * bh + 1, :]
        m = (d * 8 + t).astype(jnp.float32)
        parts.append(jax.lax.dot_general(
            m, ident, dnT, preferred_element_type=jnp.float32))
    out_ref[...] = jnp.concatenate(parts, axis=0).astype(jnp.int32)


def _make_cidx(daytime):
    dt3 = daytime.reshape(B // 128, 128, L, 2).transpose(2, 0, 3, 1) \
                 .reshape(L, 2 * B // 128, 128)
    return pl.pallas_call(
        _cidx_kernel,
        grid=(BHG,),
        in_specs=[pl.BlockSpec((L, 8, 128), lambda i: (0, i, 0))],
        out_specs=pl.BlockSpec((BB, L), lambda i: (i, 0)),
        out_shape=jax.ShapeDtypeStruct((B, L), jnp.int32),
        compiler_params=pltpu.CompilerParams(
            dimension_semantics=("arbitrary",),
        ),
    )(dt3)


NW = 32
BPW = B // NW
CBB = 2
NCHUNK = BPW // CBB
NGRP = 13

_SPLAT_DNUMS = jax.lax.GatherDimensionNumbers(
    offset_dims=(), collapsed_slice_dims=(0,), start_index_map=(0,))


def _expand_chunk(tab_v, idx_v, out_v):
    lane = jax.lax.iota(jnp.int32, 16)
    for bq in range(CBB):
        bqv = jnp.full((16,), bq, jnp.int32)

        def group(g, carry):
            off = jnp.minimum(16 * g, L - 16)
            raw = idx_v[bq, pl.ds(off, 16)]
            bases = (raw & 63) * 96
            toks = off + lane
            for c in range(OUT):
                vals = plsc.load_gather(tab_v, [bases + c])
                plsc.store_scatter(
                    out_v, [bqv, toks, jnp.full((16,), c, jnp.int32)], vals)
            return carry
        lax.fori_loop(0, NGRP, group, 0)


def _sc_body(cidx_hbm, ctab_hbm, out_hbm,
             tab_v, idx_v0, idx_v1, out_v0, out_v1,
             isem0, isem1, osem0, osem1):
    wid = lax.axis_index("s") * 2 + lax.axis_index("c")
    b0 = wid * BPW

    pltpu.sync_copy(ctab_hbm, tab_v)
    idx = (idx_v0, idx_v1)
    out = (out_v0, out_v1)
    isem = (isem0, isem1)
    osem = (osem0, osem1)

    pltpu.async_copy(cidx_hbm.at[pl.ds(b0, CBB)], idx_v0, isem0)

    def body2(c2, carry):
        for k in range(2):
            c = 2 * c2 + k
            base = b0 + c * CBB
            pltpu.make_async_copy(
                cidx_hbm.at[pl.ds(b0, CBB)], idx[k], isem[k]).wait()
            @pl.when(c + 1 < NCHUNK)
            def _():
                pltpu.async_copy(
                    cidx_hbm.at[pl.ds(base + CBB, CBB)], idx[1 - k],
                    isem[1 - k])
            @pl.when(c >= 2)
            def _():
                pltpu.make_async_copy(
                    out[k], out_hbm.at[pl.ds(b0, CBB)], osem[k]).wait()
            _expand_chunk(tab_v, idx[k], out[k])
            pltpu.async_copy(out[k], out_hbm.at[pl.ds(base, CBB)], osem[k])
        return carry

    lax.fori_loop(0, NCHUNK // 2, body2, 0)
    pltpu.make_async_copy(out_v0, out_hbm.at[pl.ds(b0, CBB)], osem0).wait()
    pltpu.make_async_copy(out_v1, out_hbm.at[pl.ds(b0, CBB)], osem1).wait()


def _sc_expand(cidx, ctab_flat):
    mesh = plsc.VectorSubcoreMesh(core_axis_name="c", subcore_axis_name="s")
    run = pl.kernel(
        _sc_body,
        mesh=mesh,
        compiler_params=pltpu.CompilerParams(needs_layout_passes=False),
        out_type=jax.ShapeDtypeStruct((B, L, OUT), jnp.float32),
        scratch_types=[
            pltpu.VMEM((64 * 96,), jnp.float32),
            pltpu.VMEM((CBB, L), jnp.int32),
            pltpu.VMEM((CBB, L), jnp.int32),
            pltpu.VMEM((CBB, L, OUT), jnp.float32),
            pltpu.VMEM((CBB, L, OUT), jnp.float32),
            pltpu.SemaphoreType.DMA,
            pltpu.SemaphoreType.DMA,
            pltpu.SemaphoreType.DMA,
            pltpu.SemaphoreType.DMA,
        ],
    )
    return run(cidx, ctab_flat)


def kernel(daytime, embedding_day, embedding_time):
    cidx = _make_cidx(daytime)
    dpad = jnp.pad(embedding_day, ((0, 1), (0, 0)))
    tpad = jnp.pad(embedding_time[:7], ((0, 1), (0, 0)))
    ctab = jnp.concatenate(
        [jnp.broadcast_to(dpad[:, None, :], (8, 8, DAY_SIZE)),
         jnp.broadcast_to(tpad[None, :, :], (8, 8, TIME_SIZE))],
        axis=-1).reshape(64 * OUT)
    return _sc_expand(cidx, ctab)


# speedup vs baseline: 7.0825x; 7.0825x over previous
import jax
import jax.numpy as jnp
from jax import lax
from jax.experimental import pallas as pl
from jax.experimental.pallas import tpu as pltpu
from jax.experimental.pallas import tpu_sc as plsc

B, L = 16384, 200
DAY_SIZE, TIME_SIZE = 32, 64
OUT = DAY_SIZE + TIME_SIZE
N = B * L
LANES = 128
ROWS = N // LANES

NW = 32
GPW = (B // 128) // NW
NGRP = 13


def _sc_transpose_body(dt_hbm, cidx_hbm, in_v, cidx_v, tbuf_v):
    wid = lax.axis_index("s") * 2 + lax.axis_index("c")
    lane = jax.lax.iota(jnp.int32, 16)

    for q in range(GPW):
        g = wid * GPW + q
        pltpu.sync_copy(dt_hbm.at[:, pl.ds(2 * g, 2), :], in_v)

        def merge(l, carry):
            for k in range(LANES // 16):
                s = pl.ds(16 * k, 16)
                cidx_v[l, s] = in_v[l, 0, s] * 8 + in_v[l, 1, s]
            return carry
        lax.fori_loop(0, L, merge, 0)

        def batch(j, carry):
            jv = jnp.full((16,), j, jnp.int32)

            def grp(r, c2):
                off = jnp.minimum(16 * r, L - 16)
                vals = plsc.load_gather(cidx_v, [off + lane, jv])
                tbuf_v[pl.ds(j * L + off, 16)] = vals
                return c2
            lax.fori_loop(0, NGRP, grp, 0)
            return carry
        lax.fori_loop(0, LANES, batch, 0)

        pltpu.sync_copy(tbuf_v, cidx_hbm.at[pl.ds(g * 128 * L, 128 * L)])


def _make_cidx_sc(daytime):
    dt3 = daytime.reshape(B // 128, 128, L, 2).transpose(2, 0, 3, 1) \
                 .reshape(L, 2 * B // 128, 128)
    mesh = plsc.VectorSubcoreMesh(core_axis_name="c", subcore_axis_name="s")
    run = pl.kernel(
        _sc_transpose_body,
        mesh=mesh,
        compiler_params=pltpu.CompilerParams(needs_layout_passes=False),
        out_type=jax.ShapeDtypeStruct((N,), jnp.int32),
        scratch_types=[
            pltpu.VMEM((L, 2, LANES), jnp.int32),
            pltpu.VMEM((L, LANES), jnp.int32),
            pltpu.VMEM((128 * L,), jnp.int32),
        ],
    )
    return run(dt3)


BLK = 64
GRID = ROWS // BLK


def _embed_kernel(cidx_ref, ctab_ref, out_ref):
    cidx = cidx_ref[...]
    iota = jax.lax.broadcasted_iota(jnp.int32, (BLK, LANES, 64), 2)
    onehot = (cidx[:, :, None] == iota).astype(jnp.float32)
    dn = (((2,), (0,)), ((), ()))
    out_ref[...] = jax.lax.dot_general(
        onehot, ctab_ref[...], dn, preferred_element_type=jnp.float32)


def kernel(daytime, embedding_day, embedding_time):
    cidx = _make_cidx_sc(daytime).reshape(ROWS, LANES)
    dpad = jnp.pad(embedding_day, ((0, 1), (0, 0)))
    tpad = jnp.pad(embedding_time[:7], ((0, 1), (0, 0)))
    ctab = jnp.concatenate(
        [jnp.broadcast_to(dpad[:, None, :], (8, 8, DAY_SIZE)),
         jnp.broadcast_to(tpad[None, :, :], (8, 8, TIME_SIZE))],
        axis=-1).reshape(64, OUT)

    out = pl.pallas_call(
        _embed_kernel,
        grid=(GRID,),
        in_specs=[
            pl.BlockSpec((BLK, LANES), lambda i: (i, 0)),
            pl.BlockSpec((64, OUT), lambda i: (0, 0)),
        ],
        out_specs=pl.BlockSpec((BLK, LANES, OUT), lambda i: (i, 0, 0)),
        out_shape=jax.ShapeDtypeStruct((ROWS, LANES, OUT), jnp.float32),
        compiler_params=pltpu.CompilerParams(
            dimension_semantics=("arbitrary",),
        ),
    )(cidx, ctab)
    return out.reshape(B, L, OUT)
